# baseline (speedup 1.0000x reference)
import functools

import jax
import jax.numpy as jnp
from jax import lax
from jax.experimental import pallas as pl
from jax.experimental.pallas import tpu as pltpu
from jax.experimental.pallas import tpu_sc as plsc

N = 10000
E = 320000
D = 128

NC = 2
NS = 16
L = 16
NW = NC * NS

CK = 128
TCH = 2560
CH0 = 160
GC = 40
EPAD = TCH * CK
ROWS_T = 624
TAIL_BASE = NS * ROWS_T
TAIL_ROWS = N - TAIL_BASE


def _sc_agg_body(feat_hbm, src_hbm, dst_hbm, w_hbm, out_hbm,
                 src_v, dst_v, w_v, rowbuf, rowbuf1, agg_sh, sem, sem1):
    c = lax.axis_index("c")
    s = lax.axis_index("s")
    start = s * CH0
    ngrp = jnp.where(c == 0, CH0 // GC, 0)
    base = s * ROWS_T

    @pl.when(c == 0)
    def _():
        def zrow(i, carry):
            for jj in range(D // L):
                rowbuf[i, pl.ds(jj * L, L)] = jnp.zeros((L,), jnp.float32)
            return carry
        lax.fori_loop(0, CK, zrow, 0)
        off = 0
        for nrows in (128, 128, 128, 128, 112):
            pltpu.sync_copy(rowbuf.at[pl.ds(0, nrows)],
                            agg_sh.at[pl.ds(base + off, nrows)])
            off += nrows

        @pl.when(s == NS - 1)
        def _():
            pltpu.sync_copy(rowbuf.at[pl.ds(0, TAIL_ROWS)],
                            agg_sh.at[pl.ds(TAIL_BASE, TAIL_ROWS)])
    plsc.subcore_barrier()

    def scale_and_scatter(gi, buf):
        @plsc.parallel_loop(0, CK, unroll=8)
        def _(i):
            wv = plsc.load_gather(
                w_v, (jnp.full((L,), gi * CK + i, jnp.int32),))
            for jj in range(D // L):
                sl = pl.ds(jj * L, L)
                buf[i, sl] = buf[i, sl] * wv
        pltpu.sync_copy(buf, agg_sh.at[dst_v.at[gi]], add=True)

    def group_body(grp, carry):
        cb = start + grp * GC
        pltpu.sync_copy(src_hbm.at[pl.ds(cb, GC)], src_v)
        pltpu.sync_copy(dst_hbm.at[pl.ds(cb, GC)], dst_v)
        pltpu.sync_copy(w_hbm.at[pl.ds(cb * CK, GC * CK)], w_v)
        pltpu.async_copy(feat_hbm.at[src_v.at[0]], rowbuf, sem)

        def chunk_pair(t, ccarry):
            g0 = 2 * t
            pltpu.async_copy(feat_hbm.at[src_v.at[g0 + 1]], rowbuf1, sem1)
            pltpu.make_async_copy(
                feat_hbm.at[src_v.at[g0]], rowbuf, sem).wait()
            scale_and_scatter(g0, rowbuf)

            @pl.when(g0 + 2 < GC)
            def _():
                pltpu.async_copy(feat_hbm.at[src_v.at[g0 + 2]], rowbuf, sem)
            pltpu.make_async_copy(
                feat_hbm.at[src_v.at[g0 + 1]], rowbuf1, sem1).wait()
            scale_and_scatter(g0 + 1, rowbuf1)
            return ccarry
        lax.fori_loop(0, GC // 2, chunk_pair, 0)
        return carry
    lax.fori_loop(0, ngrp, group_body, 0)
    plsc.subcore_barrier()

    @pl.when(c == 0)
    def _():
        pltpu.sync_copy(agg_sh.at[pl.ds(base, ROWS_T)],
                        out_hbm.at[pl.ds(base, ROWS_T)])

        @pl.when(s == NS - 1)
        def _():
            pltpu.sync_copy(agg_sh.at[pl.ds(TAIL_BASE, TAIL_ROWS)],
                            out_hbm.at[pl.ds(TAIL_BASE, TAIL_ROWS)])


_sc_aggregate = functools.partial(
    pl.kernel,
    out_type=jax.ShapeDtypeStruct((N, D), jnp.float32),
    mesh=plsc.VectorSubcoreMesh(
        core_axis_name="c", subcore_axis_name="s",
        num_cores=NC, num_subcores=NS),
    scratch_types=[
        pltpu.VMEM((GC, CK), jnp.int32),
        pltpu.VMEM((GC, CK), jnp.int32),
        pltpu.VMEM((GC * CK,), jnp.float32),
        pltpu.VMEM((CK, D), jnp.float32),
        pltpu.VMEM((CK, D), jnp.float32),
        pltpu.VMEM_SHARED((N, D), jnp.float32),
        pltpu.SemaphoreType.DMA,
        pltpu.SemaphoreType.DMA,
    ],
    compiler_params=pltpu.CompilerParams(needs_layout_passes=False),
)(_sc_agg_body)


BR = 1000
G = N // BR


def _tc_head_body(p_ref, w_ref, b_ref, a1_ref, h_ref, stats_ref):
    i = pl.program_id(0)
    x = p_ref[...]
    h = jnp.dot(x, w_ref[...], preferred_element_type=jnp.float32) + b_ref[...]
    h = jnp.where(h >= 0, h, h * a1_ref[...])
    h_ref[...] = h

    @pl.when(i == 0)
    def _():
        stats_ref[...] = jnp.zeros_like(stats_ref)

    stats_ref[0:1, :] += jnp.sum(h, axis=0, keepdims=True)
    stats_ref[1:2, :] += jnp.sum(h * h, axis=0, keepdims=True)


def _tc_tail_body(h_ref, stats_ref, g_ref, be_ref, a2_ref, o_ref):
    mean = stats_ref[0:1, :] * (1.0 / N)
    ex2 = stats_ref[1:2, :] * (1.0 / N)
    var = ex2 - mean * mean
    inv = lax.rsqrt(var + 1e-5)
    t = (h_ref[...] - mean) * (inv * g_ref[...]) + be_ref[...]
    o_ref[...] = jnp.where(t >= 0, t, t * a2_ref[...])


def kernel(feat, edge_index, edge_weight, W, b, prelu1_a, bn_gamma, bn_beta,
           prelu2_a):
    pad = EPAD - E
    src = jnp.concatenate([edge_index[0], jnp.zeros((pad,), jnp.int32)])
    dst = jnp.concatenate([edge_index[1], jnp.zeros((pad,), jnp.int32)])
    ew = jnp.concatenate([edge_weight, jnp.zeros((pad,), jnp.float32)])
    src = src.reshape(TCH, CK)
    dst = dst.reshape(TCH, CK)

    agg = _sc_aggregate(feat, src, dst, ew)

    row = lambda v: jnp.broadcast_to(v.reshape(1, -1), (1, D))
    h, stats = pl.pallas_call(
        _tc_head_body,
        grid=(G,),
        in_specs=[
            pl.BlockSpec((BR, D), lambda i: (i, 0)),
            pl.BlockSpec((D, D), lambda i: (0, 0)),
            pl.BlockSpec((1, D), lambda i: (0, 0)),
            pl.BlockSpec((1, D), lambda i: (0, 0)),
        ],
        out_specs=[
            pl.BlockSpec((BR, D), lambda i: (i, 0)),
            pl.BlockSpec((8, D), lambda i: (0, 0)),
        ],
        out_shape=[
            jax.ShapeDtypeStruct((N, D), jnp.float32),
            jax.ShapeDtypeStruct((8, D), jnp.float32),
        ],
    )(agg, W, b.reshape(1, D), row(prelu1_a))

    out = pl.pallas_call(
        _tc_tail_body,
        grid=(G,),
        in_specs=[
            pl.BlockSpec((BR, D), lambda i: (i, 0)),
            pl.BlockSpec((8, D), lambda i: (0, 0)),
            pl.BlockSpec((1, D), lambda i: (0, 0)),
            pl.BlockSpec((1, D), lambda i: (0, 0)),
            pl.BlockSpec((1, D), lambda i: (0, 0)),
        ],
        out_specs=pl.BlockSpec((BR, D), lambda i: (i, 0)),
        out_shape=jax.ShapeDtypeStruct((N, D), jnp.float32),
    )(h, stats, row(bn_gamma), row(bn_beta), row(prelu2_a))
    return out


# speedup vs baseline: 3.1096x; 3.1096x over previous
import functools

import jax
import jax.numpy as jnp
from jax import lax
from jax.experimental import pallas as pl
from jax.experimental.pallas import tpu as pltpu
from jax.experimental.pallas import tpu_sc as plsc

N = 10000
E = 320000
D = 128

NC = 2
NS = 16
L = 16
NW = NC * NS

CK = 128
TCH = 2560
CH = 80
GC = 40
EPAD = TCH * CK
ROWS_T = 624
TAIL_BASE = NS * ROWS_T
TAIL_ROWS = N - TAIL_BASE


def _sc_agg_body(feat_hbm, src_hbm, dst_hbm, w_hbm, out_hbm,
                 src_v, dst_v, w_v, rowbuf, rowbuf1, agg_sh, sem, sem1):
    c = lax.axis_index("c")
    s = lax.axis_index("s")
    start = (c * NS + s) * CH
    base = s * ROWS_T

    def zrow(i, carry):
        for jj in range(D // L):
            rowbuf[i, pl.ds(jj * L, L)] = jnp.zeros((L,), jnp.float32)
        return carry
    lax.fori_loop(0, CK, zrow, 0)
    off = 0
    for nrows in (128, 128, 128, 128, 112):
        pltpu.sync_copy(rowbuf.at[pl.ds(0, nrows)],
                        agg_sh.at[pl.ds(base + off, nrows)])
        off += nrows

    @pl.when(s == NS - 1)
    def _():
        pltpu.sync_copy(rowbuf.at[pl.ds(0, TAIL_ROWS)],
                        agg_sh.at[pl.ds(TAIL_BASE, TAIL_ROWS)])
    plsc.subcore_barrier()

    def scale_and_scatter(gi, buf):
        @plsc.parallel_loop(0, CK, unroll=8)
        def _(i):
            wv = plsc.load_gather(
                w_v, (jnp.full((L,), gi * CK + i, jnp.int32),))
            for jj in range(D // L):
                sl = pl.ds(jj * L, L)
                buf[i, sl] = buf[i, sl] * wv
        pltpu.sync_copy(buf, agg_sh.at[dst_v.at[gi]], add=True)

    def group_body(grp, carry):
        cb = start + grp * GC
        pltpu.sync_copy(src_hbm.at[pl.ds(cb, GC)], src_v)
        pltpu.sync_copy(dst_hbm.at[pl.ds(cb, GC)], dst_v)
        pltpu.sync_copy(w_hbm.at[pl.ds(cb * CK, GC * CK)], w_v)
        pltpu.async_copy(feat_hbm.at[src_v.at[0]], rowbuf, sem)

        def chunk_pair(t, ccarry):
            g0 = 2 * t
            pltpu.async_copy(feat_hbm.at[src_v.at[g0 + 1]], rowbuf1, sem1)
            pltpu.make_async_copy(
                feat_hbm.at[src_v.at[g0]], rowbuf, sem).wait()
            scale_and_scatter(g0, rowbuf)

            @pl.when(g0 + 2 < GC)
            def _():
                pltpu.async_copy(feat_hbm.at[src_v.at[g0 + 2]], rowbuf, sem)
            pltpu.make_async_copy(
                feat_hbm.at[src_v.at[g0 + 1]], rowbuf1, sem1).wait()
            scale_and_scatter(g0 + 1, rowbuf1)
            return ccarry
        lax.fori_loop(0, GC // 2, chunk_pair, 0)
        return carry
    lax.fori_loop(0, CH // GC, group_body, 0)
    plsc.subcore_barrier()

    pltpu.sync_copy(agg_sh.at[pl.ds(base, ROWS_T)],
                    out_hbm.at[pl.ds(c * N + base, ROWS_T)])

    @pl.when(s == NS - 1)
    def _():
        pltpu.sync_copy(agg_sh.at[pl.ds(TAIL_BASE, TAIL_ROWS)],
                        out_hbm.at[pl.ds(c * N + TAIL_BASE, TAIL_ROWS)])


_sc_aggregate = functools.partial(
    pl.kernel,
    out_type=jax.ShapeDtypeStruct((2 * N, D), jnp.float32),
    mesh=plsc.VectorSubcoreMesh(
        core_axis_name="c", subcore_axis_name="s",
        num_cores=NC, num_subcores=NS),
    scratch_types=[
        pltpu.VMEM((GC, CK), jnp.int32),
        pltpu.VMEM((GC, CK), jnp.int32),
        pltpu.VMEM((GC * CK,), jnp.float32),
        pltpu.VMEM((CK, D), jnp.float32),
        pltpu.VMEM((CK, D), jnp.float32),
        pltpu.VMEM_SHARED((N, D), jnp.float32),
        pltpu.SemaphoreType.DMA,
        pltpu.SemaphoreType.DMA,
    ],
    compiler_params=pltpu.CompilerParams(needs_layout_passes=False),
)(_sc_agg_body)


BR = 1000
G = N // BR


def _tc_head_body(p_ref, w_ref, b_ref, a1_ref, h_ref, stats_ref):
    i = pl.program_id(0)
    x = p_ref[0] + p_ref[1]
    h = jnp.dot(x, w_ref[...], preferred_element_type=jnp.float32) + b_ref[...]
    h = jnp.where(h >= 0, h, h * a1_ref[...])
    h_ref[...] = h

    @pl.when(i == 0)
    def _():
        stats_ref[...] = jnp.zeros_like(stats_ref)

    stats_ref[0:1, :] += jnp.sum(h, axis=0, keepdims=True)
    stats_ref[1:2, :] += jnp.sum(h * h, axis=0, keepdims=True)


def _tc_tail_body(h_ref, stats_ref, g_ref, be_ref, a2_ref, o_ref):
    mean = stats_ref[0:1, :] * (1.0 / N)
    ex2 = stats_ref[1:2, :] * (1.0 / N)
    var = ex2 - mean * mean
    inv = lax.rsqrt(var + 1e-5)
    t = (h_ref[...] - mean) * (inv * g_ref[...]) + be_ref[...]
    o_ref[...] = jnp.where(t >= 0, t, t * a2_ref[...])


def kernel(feat, edge_index, edge_weight, W, b, prelu1_a, bn_gamma, bn_beta,
           prelu2_a):
    pad = EPAD - E
    spread = (jnp.arange(pad, dtype=jnp.int32) * 8) % N
    src = jnp.concatenate([edge_index[0], spread])
    dst = jnp.concatenate([edge_index[1], spread])
    ew = jnp.concatenate([edge_weight, jnp.zeros((pad,), jnp.float32)])
    src = src.reshape(TCH, CK)
    dst = dst.reshape(TCH, CK)

    partials = _sc_aggregate(feat, src, dst, ew).reshape(2, N, D)

    row = lambda v: jnp.broadcast_to(v.reshape(1, -1), (1, D))
    h, stats = pl.pallas_call(
        _tc_head_body,
        grid=(G,),
        in_specs=[
            pl.BlockSpec((2, BR, D), lambda i: (0, i, 0)),
            pl.BlockSpec((D, D), lambda i: (0, 0)),
            pl.BlockSpec((1, D), lambda i: (0, 0)),
            pl.BlockSpec((1, D), lambda i: (0, 0)),
        ],
        out_specs=[
            pl.BlockSpec((BR, D), lambda i: (i, 0)),
            pl.BlockSpec((8, D), lambda i: (0, 0)),
        ],
        out_shape=[
            jax.ShapeDtypeStruct((N, D), jnp.float32),
            jax.ShapeDtypeStruct((8, D), jnp.float32),
        ],
    )(partials, W, b.reshape(1, D), row(prelu1_a))

    out = pl.pallas_call(
        _tc_tail_body,
        grid=(G,),
        in_specs=[
            pl.BlockSpec((BR, D), lambda i: (i, 0)),
            pl.BlockSpec((8, D), lambda i: (0, 0)),
            pl.BlockSpec((1, D), lambda i: (0, 0)),
            pl.BlockSpec((1, D), lambda i: (0, 0)),
            pl.BlockSpec((1, D), lambda i: (0, 0)),
        ],
        out_specs=pl.BlockSpec((BR, D), lambda i: (i, 0)),
        out_shape=jax.ShapeDtypeStruct((N, D), jnp.float32),
    )(h, stats, row(bn_gamma), row(bn_beta), row(prelu2_a))
    return out
